# baseline (speedup 1.0000x reference)
import jax
import jax.numpy as jnp
from jax.experimental import pallas as pl
from jax.experimental.pallas import tpu as pltpu

_N = 50000
_F = 256
_K = 2048
_ROWS = 32
_PAD = _ROWS * _K


def _stage(v, ix, d, want_desc):
    c = jax.lax.broadcasted_iota(jnp.int32, v.shape, 1)
    low = (c & d) == 0
    pv = jnp.where(low, jnp.roll(v, -d, axis=1), jnp.roll(v, d, axis=1))
    pi = jnp.where(low, jnp.roll(ix, -d, axis=1), jnp.roll(ix, d, axis=1))
    gt = (v > pv) | ((v == pv) & (ix < pi))
    keep = gt == (want_desc == low)
    return jnp.where(keep, v, pv), jnp.where(keep, ix, pi)


def _topk(v):
    r_io = jax.lax.broadcasted_iota(jnp.int32, (_ROWS, _K), 0)
    c_io = jax.lax.broadcasted_iota(jnp.int32, (_ROWS, _K), 1)
    ix = r_io * _K + c_io

    row_desc = r_io < (_ROWS // 2)
    k = 2
    while k <= _K:
        d = k // 2
        while d >= 1:
            blk = (c_io & k) == 0
            v, ix = _stage(v, ix, d, blk == row_desc)
            d //= 2
        k *= 2

    rows = _ROWS
    while rows > 1:
        h = rows // 2
        a_v, b_v = v[:h], v[h:]
        a_i, b_i = ix[:h], ix[h:]
        gt = (a_v > b_v) | ((a_v == b_v) & (a_i < b_i))
        v = jnp.where(gt, a_v, b_v)
        ix = jnp.where(gt, a_i, b_i)
        if h == 1:
            rd = jnp.full((1, _K), True)
        else:
            rd = jax.lax.broadcasted_iota(jnp.int32, (h, _K), 0) < (h // 2)
        d = _K // 2
        while d >= 1:
            v, ix = _stage(v, ix, d, rd)
            d //= 2
        rows = h
    return v, ix


def _fused_kernel(e_ref, m_ref, s_ref, o_ref, sc_ref, ix_ref, g_ref):
    pid = pl.program_id(0)

    @pl.when(pid < _ROWS)
    def _score():
        w = s_ref[...]
        inv = jax.lax.rsqrt(jnp.sum(w * w))
        s = jnp.dot(e_ref[...], w, preferred_element_type=jnp.float32) * inv
        s = s + m_ref[...]
        row = pid * _K + jax.lax.broadcasted_iota(jnp.int32, (_K, 1), 0)
        s = jnp.where(row < _N, s, -jnp.inf)
        sc_ref[pid, :] = s[:, 0]

    @pl.when(pid == _ROWS - 1)
    def _select():
        v, ix = _topk(sc_ref[...])
        ix_ref[...] = ix
        g_ref[...] = jnp.tanh(v)

    @pl.when(pid >= _ROWS)
    def _gather():
        idx = ix_ref[...]
        gate = g_ref[...]
        rows = (pid - _ROWS) * _K + jax.lax.broadcasted_iota(
            jnp.int32, (_K, _K), 0)
        oh = jnp.where(rows == idx, gate, 0.0).astype(jnp.bfloat16)
        acc = jax.lax.dot_general(
            e_ref[...].astype(jnp.bfloat16), oh, (((0,), (0,)), ((), ())),
            preferred_element_type=jnp.float32)

        @pl.when(pid == _ROWS)
        def _init():
            o_ref[...] = acc

        @pl.when(pid > _ROWS)
        def _acc():
            o_ref[...] += acc


def kernel(embeddings, mask, scorer):
    e_pad = jnp.pad(embeddings, ((0, _PAD - _N), (0, 0)))
    m_pad = jnp.pad(mask, ((0, _PAD - _N), (0, 0)))

    out = pl.pallas_call(
        _fused_kernel,
        grid=(2 * _ROWS,),
        in_specs=[
            pl.BlockSpec((_K, _F), lambda i: (i % _ROWS, 0)),
            pl.BlockSpec((_K, 1), lambda i: (i % _ROWS, 0)),
            pl.BlockSpec((_F, 1), lambda i: (0, 0)),
        ],
        out_specs=pl.BlockSpec((_F, _K), lambda i: (0, 0)),
        out_shape=jax.ShapeDtypeStruct((_F, _K), jnp.float32),
        scratch_shapes=[
            pltpu.VMEM((_ROWS, _K), jnp.float32),
            pltpu.VMEM((1, _K), jnp.int32),
            pltpu.VMEM((1, _K), jnp.float32),
        ],
    )(e_pad, m_pad, scorer)
    return out
